# baseline (speedup 1.0000x reference)
import jax
import jax.numpy as jnp
from jax import lax
from jax.experimental import pallas as pl
from jax.experimental.pallas import tpu as pltpu
from jax.experimental.pallas import tpu_sc as plsc

DELTA = 0.1
EPS = 1e-10

_B, _H, _W, _C = 16, 384, 384, 2000
_L = 16
_NJ = 63
_CPAD = 1024
_GCH = _CPAD // 128


def _whdr_body(table, comps, ncmp, out,
               comp_v, nc_v, idx1, idx2, vals1, vals2, red_v, sem):
    core = lax.axis_index("c")
    b = lax.axis_index("s")
    iota = lax.iota(jnp.int32, _L)
    zeros_i = jnp.zeros((_L,), jnp.int32)

    pltpu.sync_copy(comps.at[pl.ds(b * _C * 6, _C * 6)], comp_v)
    pltpu.sync_copy(ncmp, nc_v)
    base = b * _H * _W

    def idx_block(g, _):
        for i in range(8):
            j = g * 8 + i
            k = 2 * j + core
            row = iota + k * _L
            rowc = jnp.minimum(row, _C - 1)
            ok = row < _C
            rc6 = rowc * 6
            x1 = plsc.load_gather(comp_v, [rc6])
            y1 = plsc.load_gather(comp_v, [rc6 + 1])
            x2 = plsc.load_gather(comp_v, [rc6 + 2])
            y2 = plsc.load_gather(comp_v, [rc6 + 3])
            i1 = jnp.where(ok, base + y1 * _W + x1, 0)
            i2 = jnp.where(ok, base + y2 * _W + x2, 0)
            idx1[g, pl.ds(i * _L, _L)] = i1
            idx2[g, pl.ds(i * _L, _L)] = i2
        pltpu.async_copy(table.at[idx1.at[g]],
                         vals1.at[pl.ds(g * 128, 128)], sem)
        pltpu.async_copy(table.at[idx2.at[g]],
                         vals2.at[pl.ds(g * 128, 128)], sem)
        return 0

    lax.fori_loop(0, _GCH, idx_block, 0)

    def drain(g, _):
        pltpu.make_async_copy(table.at[idx1.at[0]],
                              vals1.at[pl.ds(0, 128)], sem).wait()
        pltpu.make_async_copy(table.at[idx2.at[0]],
                              vals2.at[pl.ds(0, 128)], sem).wait()
        return 0

    lax.fori_loop(0, _GCH, drain, 0)

    nb = plsc.load_gather(nc_v, [zeros_i + b])

    def acc_block(j, carry):
        am, aw = carry
        k = 2 * j + core
        row = iota + k * _L
        rowc = jnp.minimum(row, _C - 1)
        r1 = vals1[pl.ds(j * _L, _L)]
        r2 = vals2[pl.ds(j * _L, _L)]
        rc6 = rowc * 6
        dk = plsc.load_gather(comp_v, [rc6 + 4])
        wt = plsc.load_gather(comp_v, [rc6 + 5])
        wf = wt.astype(jnp.float32)
        alg = jnp.where(r2 / (r1 + EPS) > 1.0 + DELTA, 1,
                        jnp.where(r1 / (r2 + EPS) > 1.0 + DELTA, 2, 0))
        valid = row < nb
        mism = (alg != dk) & valid
        am = am + jnp.where(mism, wf, 0.0)
        aw = aw + jnp.where(valid, wf, 0.0)
        return am, aw

    z = jnp.zeros((_L,), jnp.float32)
    am, aw = lax.fori_loop(0, _NJ, acc_block, (z, z))
    red_v[...] = jnp.where(iota == 0, jnp.sum(am),
                           jnp.where(iota == 1, jnp.sum(aw),
                                     jnp.float32(0.0)))
    pltpu.sync_copy(red_v, out.at[2 * b + core])


def _mean_body(r_ref, o_ref):
    x = r_ref[...].reshape(_B, 2, _L)
    am = x[:, 0, 0] + x[:, 1, 0]
    aw = x[:, 0, 1] + x[:, 1, 1]
    o_ref[...] = jnp.mean(am / aw).reshape(1, 1)


def kernel(v_input, comparisons, numComparisons):
    table = v_input.reshape(_B * _H * _W)
    comps = comparisons.reshape(_B * _C * 6)
    mesh = plsc.VectorSubcoreMesh(core_axis_name="c", subcore_axis_name="s")
    sc_fn = pl.kernel(
        _whdr_body,
        out_type=jax.ShapeDtypeStruct((2 * _B, _L), jnp.float32),
        mesh=mesh,
        compiler_params=pltpu.CompilerParams(
            needs_layout_passes=False, use_tc_tiling_on_sc=False),
        scratch_types=[
            pltpu.VMEM((_C * 6,), jnp.int32),
            pltpu.VMEM((_B,), jnp.int32),
            pltpu.VMEM((_GCH, 128), jnp.int32),
            pltpu.VMEM((_GCH, 128), jnp.int32),
            pltpu.VMEM((_CPAD,), jnp.float32),
            pltpu.VMEM((_CPAD,), jnp.float32),
            pltpu.VMEM((_L,), jnp.float32),
            pltpu.SemaphoreType.DMA,
        ],
    )
    partials = sc_fn(table, comps, numComparisons)
    total = pl.pallas_call(
        _mean_body,
        out_shape=jax.ShapeDtypeStruct((1, 1), jnp.float32),
    )(partials)
    return total.reshape(1)


# speedup vs baseline: 1.0843x; 1.0843x over previous
import jax
import jax.numpy as jnp
from jax import lax
from jax.experimental import pallas as pl
from jax.experimental.pallas import tpu as pltpu
from jax.experimental.pallas import tpu_sc as plsc

DELTA = 0.1
EPS = 1e-10

_B, _H, _W, _C = 16, 384, 384, 2000
_L = 16
_NJ = 63
_CPAD = 1024
_GCH = _CPAD // 128


def _whdr_body(table, comps, ncmp, out,
               comp_v, nc_v, idx1, idx2, vals1, vals2, red_v, sem):
    core = lax.axis_index("c")
    b = lax.axis_index("s")
    iota = lax.iota(jnp.int32, _L)
    zeros_i = jnp.zeros((_L,), jnp.int32)

    pltpu.sync_copy(comps.at[b], comp_v)
    pltpu.sync_copy(ncmp, nc_v)
    base = b * _H * _W

    def idx_block(g, _):
        for i in range(8):
            j = g * 8 + i
            k = 2 * j + core
            row = iota + k * _L
            rowc = jnp.minimum(row, _C - 1)
            ok = row < _C
            rc6 = rowc * 6
            x1 = plsc.load_gather(comp_v, [rc6])
            y1 = plsc.load_gather(comp_v, [rc6 + 1])
            x2 = plsc.load_gather(comp_v, [rc6 + 2])
            y2 = plsc.load_gather(comp_v, [rc6 + 3])
            i1 = jnp.where(ok, base + y1 * _W + x1, 0)
            i2 = jnp.where(ok, base + y2 * _W + x2, 0)
            idx1[g, pl.ds(i * _L, _L)] = i1
            idx2[g, pl.ds(i * _L, _L)] = i2
        pltpu.async_copy(table.at[idx1.at[g]],
                         vals1.at[pl.ds(g * 128, 128)], sem)
        pltpu.async_copy(table.at[idx2.at[g]],
                         vals2.at[pl.ds(g * 128, 128)], sem)
        return 0

    lax.fori_loop(0, _GCH, idx_block, 0)

    def drain(g, _):
        pltpu.make_async_copy(table.at[idx1.at[0]],
                              vals1.at[pl.ds(0, 128)], sem).wait()
        pltpu.make_async_copy(table.at[idx2.at[0]],
                              vals2.at[pl.ds(0, 128)], sem).wait()
        return 0

    lax.fori_loop(0, _GCH, drain, 0)

    nb = plsc.load_gather(nc_v, [zeros_i + b])

    def acc_block(j, carry):
        am, aw = carry
        k = 2 * j + core
        row = iota + k * _L
        rowc = jnp.minimum(row, _C - 1)
        r1 = vals1[pl.ds(j * _L, _L)]
        r2 = vals2[pl.ds(j * _L, _L)]
        rc6 = rowc * 6
        dk = plsc.load_gather(comp_v, [rc6 + 4])
        wt = plsc.load_gather(comp_v, [rc6 + 5])
        wf = wt.astype(jnp.float32)
        alg = jnp.where(r2 / (r1 + EPS) > 1.0 + DELTA, 1,
                        jnp.where(r1 / (r2 + EPS) > 1.0 + DELTA, 2, 0))
        valid = row < nb
        mism = (alg != dk) & valid
        am = am + jnp.where(mism, wf, 0.0)
        aw = aw + jnp.where(valid, wf, 0.0)
        return am, aw

    z = jnp.zeros((_L,), jnp.float32)
    am, aw = lax.fori_loop(0, _NJ, acc_block, (z, z))
    red_v[...] = jnp.where(iota == 0, jnp.sum(am),
                           jnp.where(iota == 1, jnp.sum(aw),
                                     jnp.float32(0.0)))
    pltpu.sync_copy(red_v, out.at[2 * b + core])


def _mean_body(r_ref, o_ref):
    x = r_ref[...].reshape(_B, 2, _L)
    am = x[:, 0, 0] + x[:, 1, 0]
    aw = x[:, 0, 1] + x[:, 1, 1]
    o_ref[...] = jnp.mean(am / aw).reshape(1, 1)


def kernel(v_input, comparisons, numComparisons):
    table = v_input.reshape(_B * _H * _W)
    comps = comparisons.reshape(_B, _C * 6)
    mesh = plsc.VectorSubcoreMesh(core_axis_name="c", subcore_axis_name="s")
    sc_fn = pl.kernel(
        _whdr_body,
        out_type=jax.ShapeDtypeStruct((2 * _B, _L), jnp.float32),
        mesh=mesh,
        compiler_params=pltpu.CompilerParams(
            needs_layout_passes=False, use_tc_tiling_on_sc=False),
        scratch_types=[
            pltpu.VMEM((_C * 6,), jnp.int32),
            pltpu.VMEM((_B,), jnp.int32),
            pltpu.VMEM((_GCH, 128), jnp.int32),
            pltpu.VMEM((_GCH, 128), jnp.int32),
            pltpu.VMEM((_CPAD,), jnp.float32),
            pltpu.VMEM((_CPAD,), jnp.float32),
            pltpu.VMEM((_L,), jnp.float32),
            pltpu.SemaphoreType.DMA,
        ],
    )
    partials = sc_fn(table, comps, numComparisons)
    total = pl.pallas_call(
        _mean_body,
        out_shape=jax.ShapeDtypeStruct((1, 1), jnp.float32),
    )(partials)
    return total.reshape(1)
